# Initial kernel scaffold; baseline (speedup 1.0000x reference)
import jax
import jax.numpy as jnp
from jax.experimental import pallas as pl


def kernel(cath_idx, n, emb_C, emb_A, emb_T):
    raise NotImplementedError("write your pallas kernel here")


# speedup vs baseline: 1.0169x; 1.0169x over previous
import jax
import jax.numpy as jnp
from jax.experimental import pallas as pl
from jax.experimental.pallas import tpu as pltpu

FOLD_EMB_DIM = 128
N_SEQ = 512
BB = 8


def _bcast_body(idx_ref, c_ref, a_ref, t_ref, out_ref):
    i = pl.program_id(0)
    for r in range(BB):
        b = i * BB + r
        ci = idx_ref[b, 0]
        ai = idx_ref[b, 1]
        ti = idx_ref[b, 2]
        c_row = c_ref[ci, :]
        a_row = a_ref[ai, :]
        t_row = t_ref[ti, :]
        out_ref[r, :, 0:128] = jnp.broadcast_to(c_row[None, :], (N_SEQ, 128))
        out_ref[r, :, 128:256] = jnp.broadcast_to(a_row[None, :], (N_SEQ, 128))
        out_ref[r, :, 256:384] = jnp.broadcast_to(t_row[None, :], (N_SEQ, 128))


def kernel(cath_idx, n, emb_C, emb_A, emb_T):
    del n
    bs = cath_idx.shape[0]
    idx = cath_idx.astype(jnp.int32)
    d = 3 * FOLD_EMB_DIM
    grid = (bs // BB,)
    out = pl.pallas_call(
        _bcast_body,
        grid=grid,
        in_specs=[
            pl.BlockSpec(memory_space=pltpu.SMEM),
            pl.BlockSpec(emb_C.shape, lambda i: (0, 0)),
            pl.BlockSpec(emb_A.shape, lambda i: (0, 0)),
            pl.BlockSpec(emb_T.shape, lambda i: (0, 0)),
        ],
        out_specs=pl.BlockSpec((BB, N_SEQ, d), lambda i: (i, 0, 0)),
        out_shape=jax.ShapeDtypeStruct((bs, N_SEQ, d), jnp.float32),
    )(idx, emb_C, emb_A, emb_T)
    return out
